# Initial kernel scaffold; baseline (speedup 1.0000x reference)
import jax
import jax.numpy as jnp
from jax.experimental import pallas as pl


def kernel(x, x_FT_index, edge_index, batch, FT_output, W1_l, W1_self, b1, W2_l, W2_self, b2, Wp, L1_W, L1_b, L2_W, L2_b):
    raise NotImplementedError("write your pallas kernel here")


# speedup vs baseline: 8.4002x; 8.4002x over previous
import functools

import jax
import jax.numpy as jnp
from jax import lax
from jax.experimental import pallas as pl
from jax.experimental.pallas import tpu as pltpu
from jax.experimental.pallas import tpu_sc as plsc

N = 10000
E = 320000
DH = 64
H = 128
P = 64
NG = 256
ANN0 = 64

NC = 2
NS = 16
NW = NC * NS
EPW = E // NW
CH = 128
NFULL = EPW // CH
EREM = EPW - NFULL * CH
RPT = N // NS

f32 = jnp.float32
i32 = jnp.int32

_mesh = plsc.VectorSubcoreMesh(core_axis_name="c", subcore_axis_name="s")
_sc_params = pltpu.CompilerParams(use_tc_tiling_on_sc=False)


@functools.partial(
    pl.kernel,
    mesh=_mesh,
    out_type=jax.ShapeDtypeStruct((N, DH), f32),
    compiler_params=_sc_params,
    scratch_types=[
        pltpu.VMEM((CH,), i32),
        pltpu.VMEM((CH, DH), f32),
        pltpu.SemaphoreType.DMA,
    ],
)
def _sc_gather_xf(ft_hbm, idx_hbm, out_hbm, idxb, rows, sem):
    c = lax.axis_index("c")
    s = lax.axis_index("s")
    wid = s * NC + c

    def run(off, cnt):
        pltpu.sync_copy(idx_hbm.at[pl.ds(off, cnt)], idxb.at[pl.ds(0, cnt)])
        cp = pltpu.make_async_copy(
            ft_hbm.at[idxb.at[pl.ds(0, cnt)]], rows.at[pl.ds(0, cnt)], sem)
        cp.start()
        cp.wait()
        pltpu.sync_copy(rows.at[pl.ds(0, cnt)], out_hbm.at[pl.ds(off, cnt)])

    base = wid * 320

    @pl.when(wid < NW - 1)
    def _():
        run(base, 128)
        run(base + 128, 128)
        run(base + 256, 64)

    @pl.when(wid == NW - 1)
    def _():
        run(base, 64)
        run(base + 64, 16)


@functools.partial(
    pl.kernel,
    mesh=_mesh,
    out_type=(jax.ShapeDtypeStruct((NC * N, DH), f32),
              jax.ShapeDtypeStruct((NC * N, DH), f32)),
    compiler_params=_sc_params,
    scratch_types=[
        pltpu.VMEM((CH,), i32),
        pltpu.VMEM((CH,), i32),
        pltpu.VMEM((CH,), i32),
        pltpu.VMEM((CH,), i32),
        pltpu.VMEM((CH, DH), f32),
        pltpu.VMEM((CH, DH), f32),
        pltpu.VMEM((CH, DH), f32),
        pltpu.VMEM((CH, DH), f32),
        pltpu.VMEM((EREM,), i32),
        pltpu.VMEM((EREM,), i32),
        pltpu.VMEM((EREM, DH), f32),
        pltpu.VMEM((EREM, DH), f32),
        pltpu.VMEM((CH, DH), f32),
        pltpu.VMEM_SHARED((N, DH), f32),
        pltpu.VMEM_SHARED((N, DH), f32),
        pltpu.SemaphoreType.DMA,
        pltpu.SemaphoreType.DMA,
        pltpu.SemaphoreType.DMA,
        pltpu.SemaphoreType.DMA,
    ],
)
def _sc_spmm(tA, tB, src_hbm, dst_hbm, outA, outB,
             srcb0, srcb1, dstb0, dstb1,
             stgA0, stgA1, stgB0, stgB1,
             srcr, dstr, stgAr, stgBr,
             zbuf, accA, accB, semA0, semA1, semB0, semB1):
    c = lax.axis_index("c")
    s = lax.axis_index("s")
    wid = s * NC + c

    zv = jnp.zeros((16,), f32)

    def zrow(i, carry):
        for j in range(DH // 16):
            zbuf[i, pl.ds(j * 16, 16)] = zv
        return carry

    lax.fori_loop(0, CH, zrow, 0)

    rbase = s * RPT
    for j in range(RPT // CH):
        pltpu.sync_copy(zbuf, accA.at[pl.ds(rbase + j * CH, CH)])
        pltpu.sync_copy(zbuf, accB.at[pl.ds(rbase + j * CH, CH)])
    rrem = RPT - (RPT // CH) * CH
    if rrem:
        off = rbase + (RPT // CH) * CH
        pltpu.sync_copy(zbuf.at[pl.ds(0, rrem)], accA.at[pl.ds(off, rrem)])
        pltpu.sync_copy(zbuf.at[pl.ds(0, rrem)], accB.at[pl.ds(off, rrem)])
    plsc.subcore_barrier()

    ebase = wid * EPW
    srcb = (srcb0, srcb1)
    dstb = (dstb0, dstb1)
    stgA = (stgA0, stgA1)
    stgB = (stgB0, stgB1)
    semA = (semA0, semA1)
    semB = (semB0, semB1)

    def load_and_fire(k, p):
        off = ebase + k * CH
        pltpu.sync_copy(src_hbm.at[pl.ds(off, CH)], srcb[p])
        pltpu.sync_copy(dst_hbm.at[pl.ds(off, CH)], dstb[p])
        pltpu.make_async_copy(tA.at[srcb[p]], stgA[p], semA[p]).start()
        pltpu.make_async_copy(tB.at[srcb[p]], stgB[p], semB[p]).start()

    def drain_and_scatter(p):
        pltpu.make_async_copy(tA.at[srcb[p]], stgA[p], semA[p]).wait()
        pltpu.make_async_copy(tB.at[srcb[p]], stgB[p], semB[p]).wait()
        pltpu.sync_copy(stgA[p], accA.at[dstb[p]], add=True)
        pltpu.sync_copy(stgB[p], accB.at[dstb[p]], add=True)

    load_and_fire(0, 0)
    load_and_fire(1, 1)

    def body(i, carry):
        for p in range(2):
            k = 2 * i + p
            drain_and_scatter(p)

            @pl.when(k + 2 < NFULL)
            def _():
                load_and_fire(k + 2, p)
        return carry

    lax.fori_loop(0, NFULL // 2, body, 0)

    roff = ebase + NFULL * CH
    pltpu.sync_copy(src_hbm.at[pl.ds(roff, EREM)], srcr)
    pltpu.sync_copy(dst_hbm.at[pl.ds(roff, EREM)], dstr)
    pltpu.make_async_copy(tA.at[srcr], stgAr, semA0).start()
    pltpu.make_async_copy(tB.at[srcr], stgBr, semB0).start()
    pltpu.make_async_copy(tA.at[srcr], stgAr, semA0).wait()
    pltpu.make_async_copy(tB.at[srcr], stgBr, semB0).wait()
    pltpu.sync_copy(stgAr, accA.at[dstr], add=True)
    pltpu.sync_copy(stgBr, accB.at[dstr], add=True)

    plsc.subcore_barrier()
    pltpu.sync_copy(accA.at[pl.ds(rbase, RPT)],
                    outA.at[pl.ds(c * N + rbase, RPT)])
    pltpu.sync_copy(accB.at[pl.ds(rbase, RPT)],
                    outB.at[pl.ds(c * N + rbase, RPT)])


BN = 1000


def _tc_layer_body(aggA_ref, aggB_ref, inA_ref, inB_ref,
                   wlA_ref, wlB_ref, wsA_ref, wsB_ref, b_ref,
                   outA_ref, outB_ref):
    aggA = aggA_ref[0] + aggA_ref[1]
    aggB = aggB_ref[0] + aggB_ref[1]
    acc = jnp.dot(aggA, wlA_ref[...], preferred_element_type=f32)
    acc += jnp.dot(aggB, wlB_ref[...], preferred_element_type=f32)
    acc += jnp.dot(inA_ref[...], wsA_ref[...], preferred_element_type=f32)
    acc += jnp.dot(inB_ref[...], wsB_ref[...], preferred_element_type=f32)
    acc += b_ref[...]
    acc = jnp.where(acc > 0, acc, 0.1 * acc)
    outA_ref[...] = acc[:, :DH]
    outB_ref[...] = acc[:, DH:]


_layer_call = pl.pallas_call(
    _tc_layer_body,
    grid=(N // BN,),
    in_specs=[
        pl.BlockSpec((2, BN, DH), lambda i: (0, i, 0)),
        pl.BlockSpec((2, BN, DH), lambda i: (0, i, 0)),
        pl.BlockSpec((BN, DH), lambda i: (i, 0)),
        pl.BlockSpec((BN, DH), lambda i: (i, 0)),
        pl.BlockSpec((DH, H), lambda i: (0, 0)),
        pl.BlockSpec((DH, H), lambda i: (0, 0)),
        pl.BlockSpec((DH, H), lambda i: (0, 0)),
        pl.BlockSpec((DH, H), lambda i: (0, 0)),
        pl.BlockSpec((1, H), lambda i: (0, 0)),
    ],
    out_specs=[
        pl.BlockSpec((BN, DH), lambda i: (i, 0)),
        pl.BlockSpec((BN, DH), lambda i: (i, 0)),
    ],
    out_shape=[
        jax.ShapeDtypeStruct((N, DH), f32),
        jax.ShapeDtypeStruct((N, DH), f32),
    ],
)


def _tc_final_body(hA_ref, hB_ref, wpA_ref, wpB_ref, batch_ref,
                   l1w_ref, l1b_ref, l2w_ref, l2b_ref, out_ref, acc_ref):
    i = pl.program_id(0)
    p = jnp.dot(hA_ref[...], wpA_ref[...], preferred_element_type=f32)
    p += jnp.dot(hB_ref[...], wpB_ref[...], preferred_element_type=f32)
    gids = lax.broadcasted_iota(i32, (BN, NG), 1)
    oh = (gids == batch_ref[...]).astype(f32)
    contrib = lax.dot_general(oh, p, (((0,), (0,)), ((), ())),
                              preferred_element_type=f32)

    @pl.when(i == 0)
    def _():
        acc_ref[...] = jnp.zeros((NG, P), f32)

    acc_ref[...] += contrib

    @pl.when(i == N // BN - 1)
    def _():
        g = acc_ref[...]
        g = jnp.where(g > 0, g, 0.1 * g)
        a1 = jnp.dot(g, l1w_ref[...], preferred_element_type=f32) + l1b_ref[...]
        a1 = jnp.maximum(a1, 0.0)
        out_ref[...] = (jnp.dot(a1, l2w_ref[...], preferred_element_type=f32)
                        + l2b_ref[...])


_final_call = pl.pallas_call(
    _tc_final_body,
    grid=(N // BN,),
    in_specs=[
        pl.BlockSpec((BN, DH), lambda i: (i, 0)),
        pl.BlockSpec((BN, DH), lambda i: (i, 0)),
        pl.BlockSpec((DH, P), lambda i: (0, 0)),
        pl.BlockSpec((DH, P), lambda i: (0, 0)),
        pl.BlockSpec((BN, 1), lambda i: (i, 0)),
        pl.BlockSpec((P, ANN0), lambda i: (0, 0)),
        pl.BlockSpec((1, ANN0), lambda i: (0, 0)),
        pl.BlockSpec((ANN0, 1), lambda i: (0, 0)),
        pl.BlockSpec((1, 1), lambda i: (0, 0)),
    ],
    out_specs=pl.BlockSpec((NG, 1), lambda i: (0, 0)),
    out_shape=jax.ShapeDtypeStruct((NG, 1), f32),
    scratch_shapes=[pltpu.VMEM((NG, P), f32)],
)


def kernel(x, x_FT_index, edge_index, batch, FT_output,
           W1_l, W1_self, b1, W2_l, W2_self, b2,
           Wp, L1_W, L1_b, L2_W, L2_b):
    x = x.astype(f32)
    ei = edge_index.astype(i32)
    src = ei[0]
    dst = ei[1]
    ft_idx = x_FT_index.astype(i32)
    batch2 = batch.astype(i32).reshape(N, 1)

    w1lT = W1_l.T
    w1sT = W1_self.T
    w2lT = W2_l.T
    w2sT = W2_self.T
    wpT = Wp.T

    xf = _sc_gather_xf(FT_output.astype(f32), ft_idx)

    agg1A, agg1B = _sc_spmm(x, xf, src, dst)
    h1A, h1B = _layer_call(
        agg1A.reshape(NC, N, DH), agg1B.reshape(NC, N, DH), x, xf,
        w1lT[:DH], w1lT[DH:], w1sT[:DH], w1sT[DH:], b1.reshape(1, H))

    agg2A, agg2B = _sc_spmm(h1A, h1B, src, dst)
    h2A, h2B = _layer_call(
        agg2A.reshape(NC, N, DH), agg2B.reshape(NC, N, DH), h1A, h1B,
        w2lT[:DH], w2lT[DH:], w2sT[:DH], w2sT[DH:], b2.reshape(1, H))

    out = _final_call(
        h2A, h2B, wpT[:DH], wpT[DH:], batch2,
        L1_W.T, L1_b.reshape(1, ANN0), L2_W.T, L2_b.reshape(1, 1))
    return out
